# Initial kernel scaffold; baseline (speedup 1.0000x reference)
import functools

import jax
import jax.numpy as jnp
from jax import lax
from jax.experimental import pallas as pl
from jax.experimental.pallas import tpu as pltpu
from jax.experimental.pallas import tpu_sc as plsc

NSUB = 16
NCORE = 2
LANES = 16
CHUNK = 128


def _mm1_body(x_ref, w_ref, o_ref):
    o_ref[0] = jnp.dot(x_ref[...], w_ref[...],
                       preferred_element_type=jnp.float32)


def _mm1(x, w):
    n, d = x.shape
    h = d // 2
    r = 2000
    assert n % r == 0
    return pl.pallas_call(
        _mm1_body,
        grid=(n // r, 2),
        in_specs=[
            pl.BlockSpec((r, d), lambda i, c: (i, 0)),
            pl.BlockSpec((d, h), lambda i, c: (0, c)),
        ],
        out_specs=pl.BlockSpec((1, r, h), lambda i, c: (c, i, 0)),
        out_shape=jax.ShapeDtypeStruct((2, n, h), jnp.float32),
    )(x, w)


def _mm2_body(a_ref, w_ref, o_ref):
    k = pl.program_id(2)
    xb = jnp.maximum(a_ref[0], 0.0)
    p = jnp.dot(xb, w_ref[...], preferred_element_type=jnp.float32)

    @pl.when(k == 0)
    def _():
        o_ref[0] = p

    @pl.when(k == 1)
    def _():
        o_ref[0] = o_ref[0] + p


def _mm2(a, w):
    _, n, h = a.shape
    r = 2000
    assert n % r == 0
    return pl.pallas_call(
        _mm2_body,
        grid=(n // r, 2, 2),
        in_specs=[
            pl.BlockSpec((1, r, h), lambda i, c, k: (k, i, 0)),
            pl.BlockSpec((h, h), lambda i, c, k: (k, c)),
        ],
        out_specs=pl.BlockSpec((1, r, h), lambda i, c, k: (c, i, 0)),
        out_shape=jax.ShapeDtypeStruct((2, n, h), jnp.float32),
    )(a, w)


def _sc_prop(hsplit, src_t, dst_t, attr_t, n):
    _, _, hdim = hsplit.shape
    ch = src_t.shape[1]
    rows_pt = n // NSUB
    zr = rows_pt // 5
    assert rows_pt % zr == 0
    nvec = hdim // LANES

    mesh = plsc.VectorSubcoreMesh(core_axis_name="c", subcore_axis_name="s")

    @functools.partial(
        pl.kernel,
        out_type=jax.ShapeDtypeStruct((NCORE, n, hdim), jnp.float32),
        mesh=mesh,
        scratch_types=[
            pltpu.VMEM((ch, CHUNK), jnp.int32),
            pltpu.VMEM((ch, CHUNK), jnp.int32),
            pltpu.VMEM((ch, CHUNK), jnp.float32),
            pltpu.VMEM((CHUNK, hdim), jnp.float32),
            pltpu.VMEM((zr, hdim), jnp.float32),
            pltpu.VMEM_SHARED((n, hdim), jnp.float32),
        ],
    )
    def k(h_hbm, src_hbm, dst_hbm, attr_hbm, out_hbm,
          src_v, dst_v, attr_v, row_v, zbuf, acc):
        c = lax.axis_index("c")
        s = lax.axis_index("s")

        zv = jnp.zeros((LANES,), jnp.float32)

        @pl.loop(0, zr)
        def _(rr):
            for t in range(nvec):
                zbuf[rr, pl.ds(t * LANES, LANES)] = zv

        for t in range(rows_pt // zr):
            pltpu.sync_copy(zbuf, acc.at[pl.ds(s * rows_pt + t * zr, zr)])

        pltpu.sync_copy(src_hbm.at[s], src_v)
        pltpu.sync_copy(dst_hbm.at[s], dst_v)
        pltpu.sync_copy(attr_hbm.at[s], attr_v)
        plsc.subcore_barrier()

        @pl.loop(0, ch)
        def _(j):
            pltpu.sync_copy(h_hbm.at[c].at[src_v.at[j]], row_v)

            @pl.loop(0, CHUNK)
            def _(e):
                av = plsc.load_gather(
                    attr_v,
                    [jnp.full((LANES,), j, jnp.int32),
                     jnp.full((LANES,), e, jnp.int32)])
                for t in range(nvec):
                    sl = pl.ds(t * LANES, LANES)
                    row_v[e, sl] = row_v[e, sl] * av

            pltpu.sync_copy(row_v, acc.at[dst_v.at[j]], add=True)

        plsc.subcore_barrier()
        pltpu.sync_copy(acc.at[pl.ds(s * rows_pt, rows_pt)],
                        out_hbm.at[c].at[pl.ds(s * rows_pt, rows_pt)])

    return k(hsplit, src_t, dst_t, attr_t)


def kernel(x, edge_index, edge_attr, batch, W1, W2):
    n, d = x.shape
    e = edge_attr.shape[0]

    src = edge_index[0].astype(jnp.int32)
    dst = edge_index[1].astype(jnp.int32)
    attr = edge_attr.astype(jnp.float32)

    grp = NSUB * CHUNK
    ep = ((e + grp - 1) // grp) * grp
    pad = ep - e
    ch = ep // grp
    src_t = jnp.pad(src, (0, pad)).reshape(NSUB, ch, CHUNK)
    dst_t = jnp.pad(dst, (0, pad)).reshape(NSUB, ch, CHUNK)
    attr_t = jnp.pad(attr, (0, pad)).reshape(NSUB, ch, CHUNK)

    h1 = _mm1(x, W1)
    a1 = _sc_prop(h1, src_t, dst_t, attr_t, n)
    h2 = _mm2(a1, W2)
    out2 = _sc_prop(h2, src_t, dst_t, attr_t, n)

    return jnp.transpose(out2, (1, 0, 2)).reshape(n, d)


# speedup vs baseline: 2.3263x; 2.3263x over previous
import dataclasses
import functools

import jax
import jax.numpy as jnp
from jax import lax
from jax.experimental import pallas as pl
from jax.experimental.pallas import tpu as pltpu
from jax.experimental.pallas import tpu_sc as plsc

NSUB = 16
NCORE = 2
LANES = 16
CHUNK = 128


def _mm1_body(x_ref, w_ref, o_ref):
    o_ref[0] = jnp.dot(x_ref[...], w_ref[...],
                       preferred_element_type=jnp.float32)


def _mm1(x, w):
    n, d = x.shape
    h = d // 2
    r = _pick_block(n)
    return pl.pallas_call(
        _mm1_body,
        grid=(n // r, 2),
        in_specs=[
            pl.BlockSpec((r, d), lambda i, c: (i, 0)),
            pl.BlockSpec((d, h), lambda i, c: (0, c)),
        ],
        out_specs=pl.BlockSpec((1, r, h), lambda i, c: (c, i, 0)),
        out_shape=jax.ShapeDtypeStruct((2, n, h), jnp.float32),
    )(x, w)


def _mm2_body(a_ref, w_ref, o_ref):
    k = pl.program_id(2)
    xb = jnp.maximum(a_ref[0], 0.0)
    p = jnp.dot(xb, w_ref[...], preferred_element_type=jnp.float32)

    @pl.when(k == 0)
    def _():
        o_ref[0] = p

    @pl.when(k == 1)
    def _():
        o_ref[0] = o_ref[0] + p


def _pick_block(n):
    return next(r for r in (2000, 2048, 1280, 1000, 800, 640, 512, 400,
                            320, 256, 128, 16, 8) if n % r == 0)


def _mm2(a, w):
    _, n, h = a.shape
    r = _pick_block(n)
    return pl.pallas_call(
        _mm2_body,
        grid=(n // r, 2, 2),
        in_specs=[
            pl.BlockSpec((1, r, h), lambda i, c, k: (k, i, 0)),
            pl.BlockSpec((h, h), lambda i, c, k: (k, c)),
        ],
        out_specs=pl.BlockSpec((1, r, h), lambda i, c, k: (c, i, 0)),
        out_shape=jax.ShapeDtypeStruct((2, n, h), jnp.float32),
    )(a, w)


GBATCH = 16


def _sc_prop(hsplit, src_t, dst_t, attr_t, n_pad):
    _, _, hdim = hsplit.shape
    nb = src_t.shape[1]
    rows_pt = n_pad // NSUB
    assert rows_pt % CHUNK == 0
    nvec = hdim // LANES

    mesh = plsc.VectorSubcoreMesh(core_axis_name="c", subcore_axis_name="s")
    cp = pltpu.CompilerParams()
    if "needs_layout_passes" in pltpu.CompilerParams.__dataclass_fields__:
        cp = dataclasses.replace(cp, needs_layout_passes=False)

    @functools.partial(
        pl.kernel,
        out_type=jax.ShapeDtypeStruct((NCORE, n_pad, hdim), jnp.float32),
        mesh=mesh,
        compiler_params=cp,
        scratch_types=[
            pltpu.VMEM((GBATCH, CHUNK), jnp.int32),
            pltpu.VMEM((GBATCH, CHUNK), jnp.int32),
            pltpu.VMEM((GBATCH, CHUNK), jnp.float32),
            pltpu.VMEM((CHUNK, hdim), jnp.float32),
            pltpu.VMEM_SHARED((n_pad, hdim), jnp.float32),
        ],
    )
    def k(h_hbm, src_hbm, dst_hbm, attr_hbm, out_hbm,
          src_v, dst_v, attr_v, row_v, acc):
        c = lax.axis_index("c")
        s = lax.axis_index("s")

        zv = jnp.zeros((LANES,), jnp.float32)

        @pl.loop(0, CHUNK)
        def _(rr):
            for t in range(nvec):
                row_v[rr, pl.ds(t * LANES, LANES)] = zv

        for t in range(rows_pt // CHUNK):
            pltpu.sync_copy(row_v, acc.at[pl.ds(s * rows_pt + t * CHUNK,
                                                CHUNK)])
        plsc.subcore_barrier()

        @pl.loop(0, nb)
        def _(b):
            pltpu.sync_copy(src_hbm.at[s].at[b], src_v)
            pltpu.sync_copy(dst_hbm.at[s].at[b], dst_v)
            pltpu.sync_copy(attr_hbm.at[s].at[b], attr_v)

            @pl.loop(0, GBATCH)
            def _(j):
                pltpu.sync_copy(h_hbm.at[c].at[src_v.at[j]], row_v)

                @pl.loop(0, CHUNK)
                def _(e):
                    av = plsc.load_gather(
                        attr_v,
                        [jnp.full((LANES,), j, jnp.int32),
                         jnp.full((LANES,), e, jnp.int32)])
                    for t in range(nvec):
                        sl = pl.ds(t * LANES, LANES)
                        row_v[e, sl] = row_v[e, sl] * av

                pltpu.sync_copy(row_v, acc.at[dst_v.at[j]], add=True)

        plsc.subcore_barrier()
        pltpu.sync_copy(acc.at[pl.ds(s * rows_pt, rows_pt)],
                        out_hbm.at[c].at[pl.ds(s * rows_pt, rows_pt)])

    return k(hsplit, src_t, dst_t, attr_t)


def kernel(x, edge_index, edge_attr, batch, W1, W2):
    n, d = x.shape
    e = edge_attr.shape[0]

    src = edge_index[0].astype(jnp.int32)
    dst = edge_index[1].astype(jnp.int32)
    attr = edge_attr.astype(jnp.float32)

    grp = NSUB * GBATCH * CHUNK
    ep = ((e + grp - 1) // grp) * grp
    pad = ep - e
    nb = ep // grp
    shape = (NSUB, nb, GBATCH, CHUNK)
    src_t = jnp.pad(src, (0, pad)).reshape(shape)
    dst_t = jnp.pad(dst, (0, pad)).reshape(shape)
    attr_t = jnp.pad(attr, (0, pad)).reshape(shape)

    rows_pt = ((n + NSUB * CHUNK - 1) // (NSUB * CHUNK)) * CHUNK
    n_pad = rows_pt * NSUB

    h1 = _mm1(x, W1)
    a1 = _sc_prop(h1, src_t, dst_t, attr_t, n_pad)
    h2 = _mm2(a1, W2)
    out2 = _sc_prop(h2, src_t, dst_t, attr_t, n_pad)

    out2 = out2[:, :n, :]
    return jnp.transpose(out2, (1, 0, 2)).reshape(out2.shape[1], d)
